# baseline (speedup 1.0000x reference)
import jax
import jax.numpy as jnp
from jax import lax
from jax.experimental import pallas as pl
from jax.experimental.pallas import tpu as pltpu
from jax.experimental.pallas import tpu_sc as plsc

S = 320000
N = 10000
D = 128
NC = 2
NS = 16
LANES = 16


_MM_BS = 2560


def _mm_body(x_ref, w_ref, o_ref):
    o_ref[...] = jnp.dot(x_ref[...], w_ref[...],
                         preferred_element_type=jnp.float32)


def _matmul(x, w):
    return pl.pallas_call(
        _mm_body,
        grid=(S // _MM_BS,),
        in_specs=[
            pl.BlockSpec((_MM_BS, D), lambda i: (i, 0)),
            pl.BlockSpec((D, D), lambda i: (0, 0)),
        ],
        out_specs=pl.BlockSpec((_MM_BS, D), lambda i: (i, 0)),
        out_shape=jax.ShapeDtypeStruct((S, D), jnp.float32),
    )(x, w)


_SCAT = 80
_POOL_CHUNK = 160
_ROWS_PER_TILE = S // NS
_POOL_NCHUNK = _ROWS_PER_TILE // _POOL_CHUNK
_NP = 10240
_SEG_PER_TILE = _NP // NS


def _pool_body(x_hbm, idx_hbm, zeros_hbm, pool0_hbm, pool1_hbm,
               table, rows0, rows1, ia0, ia1, ib0, ib1,
               rsem0, rsem1, iasem0, iasem1, ibsem0, ibsem1):
    c = lax.axis_index("c")
    s = lax.axis_index("s")
    pltpu.sync_copy(zeros_hbm, table.at[pl.ds(s * _SEG_PER_TILE, _SEG_PER_TILE)])
    plsc.subcore_barrier()

    rows = (rows0, rows1)
    ia = (ia0, ia1)
    ib = (ib0, ib1)
    rsems = (rsem0, rsem1)
    iasems = (iasem0, iasem1)
    ibsems = (ibsem0, ibsem1)

    def issue(g, slot):
        base = s * _ROWS_PER_TILE + g * _POOL_CHUNK
        pltpu.async_copy(x_hbm.at[pl.ds(base, _POOL_CHUNK)], rows[slot],
                         rsems[slot])
        pltpu.async_copy(idx_hbm.at[pl.ds(c * S + base, _SCAT)],
                         ia[slot], iasems[slot])
        pltpu.async_copy(idx_hbm.at[pl.ds(c * S + base + _SCAT, _SCAT)],
                         ib[slot], ibsems[slot])

    def process(g, slot):
        pltpu.make_async_copy(x_hbm.at[pl.ds(0, _POOL_CHUNK)],
                              rows[slot], rsems[slot]).wait()
        pltpu.make_async_copy(idx_hbm.at[pl.ds(0, _SCAT)],
                              ia[slot], iasems[slot]).wait()
        pltpu.make_async_copy(idx_hbm.at[pl.ds(0, _SCAT)],
                              ib[slot], ibsems[slot]).wait()
        pltpu.sync_copy(rows[slot].at[pl.ds(0, _SCAT)],
                        table.at[ia[slot]], add=True)
        pltpu.sync_copy(rows[slot].at[pl.ds(_SCAT, _SCAT)],
                        table.at[ib[slot]], add=True)

    issue(0, 0)
    issue(1, 1)

    def pair(g2, carry):
        for slot in range(2):
            g = g2 * 2 + slot
            process(g, slot)

            @pl.when(g + 2 < _POOL_NCHUNK)
            def _():
                issue(g + 2, slot)
        return carry

    lax.fori_loop(0, _POOL_NCHUNK // 2, pair, 0)
    process(_POOL_NCHUNK - 1, 0)
    plsc.subcore_barrier()
    stripe = pl.ds(s * _SEG_PER_TILE, _SEG_PER_TILE)

    @pl.when(c == 0)
    def _():
        pltpu.sync_copy(table.at[stripe], pool0_hbm.at[stripe])

    @pl.when(c == 1)
    def _():
        pltpu.sync_copy(table.at[stripe], pool1_hbm.at[stripe])


def _pools(x, idx2, zeros):
    mesh = plsc.VectorSubcoreMesh(core_axis_name="c", subcore_axis_name="s")
    return pl.kernel(
        _pool_body,
        out_type=(
            jax.ShapeDtypeStruct((_NP, D), jnp.float32),
            jax.ShapeDtypeStruct((_NP, D), jnp.float32),
        ),
        mesh=mesh,
        scratch_types=(
            [pltpu.VMEM_SHARED((_NP, D), jnp.float32)]
            + [pltpu.VMEM((_POOL_CHUNK, D), jnp.float32)] * 2
            + [pltpu.VMEM((_SCAT,), jnp.int32)] * 4
            + [pltpu.SemaphoreType.DMA] * 6
        ),
    )(x, idx2, zeros)


def _tables_body(rp_ref, cp_ref, ncol_ref, nrow_ref, nall_ref,
                 w1_ref, w2_ref, w3_ref, p1_ref, p2_ref):
    rp = rp_ref[...][:N]
    cp = cp_ref[...][:N]
    allp = jnp.sum(rp, axis=0, keepdims=True) / nall_ref[...]
    p3 = jnp.dot(allp, w3_ref[...], preferred_element_type=jnp.float32)
    p1_ref[...] = jnp.dot(rp / ncol_ref[...], w1_ref[...],
                          preferred_element_type=jnp.float32) + p3
    p2_ref[...] = jnp.dot(cp / nrow_ref[...], w2_ref[...],
                          preferred_element_type=jnp.float32)


def _tables(row_pool, col_pool, ncol, nrow, nall, w1, w2, w3):
    return pl.pallas_call(
        _tables_body,
        out_shape=(
            jax.ShapeDtypeStruct((N, D), jnp.float32),
            jax.ShapeDtypeStruct((N, D), jnp.float32),
        ),
    )(row_pool, col_pool, ncol, nrow, nall, w1, w2, w3)


_OUT_CHUNK = 80
_ROWS_PER_WORKER = S // (NC * NS)
_OUT_NCHUNK = _ROWS_PER_WORKER // _OUT_CHUNK


def _fuse_body(t_hbm, ic_hbm, ir_hbm, p1_hbm, p2_hbm, out_hbm,
               t0, t1, t2, g10, g11, g12, g20, g21, g22,
               ic0, ic1, ic2, ir0, ir1, ir2,
               tsem0, tsem1, tsem2, icsem0, icsem1, icsem2,
               irsem0, irsem1, irsem2, g1sem0, g1sem1, g1sem2,
               g2sem0, g2sem1, g2sem2):
    c = lax.axis_index("c")
    s = lax.axis_index("s")
    w = s * NC + c
    tb = (t0, t1, t2)
    g1b = (g10, g11, g12)
    g2b = (g20, g21, g22)
    icb = (ic0, ic1, ic2)
    irb = (ir0, ir1, ir2)
    tsems = (tsem0, tsem1, tsem2)
    icsems = (icsem0, icsem1, icsem2)
    irsems = (irsem0, irsem1, irsem2)
    g1sems = (g1sem0, g1sem1, g1sem2)
    g2sems = (g2sem0, g2sem1, g2sem2)

    def issue_loads(g, slot):
        base = w * _ROWS_PER_WORKER + g * _OUT_CHUNK
        pltpu.async_copy(ic_hbm.at[pl.ds(base, _OUT_CHUNK)], icb[slot],
                         icsems[slot])
        pltpu.async_copy(ir_hbm.at[pl.ds(base, _OUT_CHUNK)], irb[slot],
                         irsems[slot])
        pltpu.async_copy(t_hbm.at[pl.ds(base, _OUT_CHUNK)], tb[slot],
                         tsems[slot])

    def wait_idx(slot):
        pltpu.make_async_copy(ic_hbm.at[pl.ds(0, _OUT_CHUNK)], icb[slot],
                              icsems[slot]).wait()
        pltpu.make_async_copy(ir_hbm.at[pl.ds(0, _OUT_CHUNK)], irb[slot],
                              irsems[slot]).wait()

    def issue_gathers(slot):
        pltpu.async_copy(p1_hbm.at[icb[slot]], g1b[slot], g1sems[slot])
        pltpu.async_copy(p2_hbm.at[irb[slot]], g2b[slot], g2sems[slot])

    def wait_idx_issue_gathers(slot):
        wait_idx(slot)
        issue_gathers(slot)

    def wait_gathers_t(slot):
        pltpu.make_async_copy(p1_hbm.at[pl.ds(0, _OUT_CHUNK)], g1b[slot],
                              g1sems[slot]).wait()
        pltpu.make_async_copy(p2_hbm.at[pl.ds(0, _OUT_CHUNK)], g2b[slot],
                              g2sems[slot]).wait()
        pltpu.make_async_copy(t_hbm.at[pl.ds(0, _OUT_CHUNK)], tb[slot],
                              tsems[slot]).wait()

    def compute_store(g, slot):
        def row(r, cc):
            for k in range(D // LANES):
                sl = pl.ds(k * LANES, LANES)
                plsc.addupdate(tb[slot].at[r, sl],
                               g1b[slot][r, sl] + g2b[slot][r, sl])
            return cc

        lax.fori_loop(0, _OUT_CHUNK, row, 0)
        base = w * _ROWS_PER_WORKER + g * _OUT_CHUNK
        pltpu.sync_copy(tb[slot], out_hbm.at[pl.ds(base, _OUT_CHUNK)])

    issue_loads(0, 0)
    issue_loads(1, 1)
    issue_loads(2, 2)
    wait_idx_issue_gathers(0)
    wait_idx_issue_gathers(1)

    def triple(g3_, carry):
        for slot in range(3):
            g = g3_ * 3 + slot
            wait_gathers_t(slot)
            compute_store(g, slot)

            @pl.when(g + 3 < _OUT_NCHUNK)
            def _():
                issue_loads(g + 3, slot)

            @pl.when(g + 2 < _OUT_NCHUNK)
            def _():
                wait_idx_issue_gathers((slot + 2) % 3)
        return carry

    lax.fori_loop(0, _OUT_NCHUNK // 3, triple, 0)
    wait_gathers_t(0)
    compute_store(_OUT_NCHUNK - 2, 0)
    wait_gathers_t(1)
    compute_store(_OUT_NCHUNK - 1, 1)


def _fuse(t, idx_col, idx_row, p1, p2):
    mesh = plsc.VectorSubcoreMesh(core_axis_name="c", subcore_axis_name="s")
    return pl.kernel(
        _fuse_body,
        out_type=jax.ShapeDtypeStruct((S, D), jnp.float32),
        mesh=mesh,
        scratch_types=(
            [pltpu.VMEM((_OUT_CHUNK, D), jnp.float32)] * 9
            + [pltpu.VMEM((_OUT_CHUNK,), jnp.int32)] * 6
            + [pltpu.SemaphoreType.DMA] * 15
        ),
    )(t, idx_col, idx_row, p1, p2)


def kernel(input_layer, idx_col, idx_col_norm, idx_row, idx_row_norm,
           idx_all, idx_all_norm, W):
    idx2 = jnp.concatenate([idx_col, idx_row])
    zeros = jnp.zeros((_SEG_PER_TILE, D), jnp.float32)
    pool0, pool1 = _pools(input_layer, idx2, zeros)
    p1, p2 = _tables(pool0, pool1,
                     idx_col_norm.reshape(N, 1), idx_row_norm.reshape(N, 1),
                     idx_all_norm.reshape(1, 1), W[1], W[2], W[3])
    t = _matmul(input_layer, W[0])
    return _fuse(t, idx_col, idx_row, p1, p2)


# speedup vs baseline: 1.0014x; 1.0014x over previous
import jax
import jax.numpy as jnp
from jax import lax
from jax.experimental import pallas as pl
from jax.experimental.pallas import tpu as pltpu
from jax.experimental.pallas import tpu_sc as plsc

S = 320000
N = 10000
D = 128
NC = 2
NS = 16
LANES = 16


_MM_BS = 2560


def _mm_body(x_ref, w_ref, o_ref):
    o_ref[...] = jnp.dot(x_ref[...], w_ref[...],
                         preferred_element_type=jnp.float32)


def _matmul(x, w):
    return pl.pallas_call(
        _mm_body,
        grid=(S // _MM_BS,),
        in_specs=[
            pl.BlockSpec((_MM_BS, D), lambda i: (i, 0)),
            pl.BlockSpec((D, D), lambda i: (0, 0)),
        ],
        out_specs=pl.BlockSpec((_MM_BS, D), lambda i: (i, 0)),
        out_shape=jax.ShapeDtypeStruct((S, D), jnp.float32),
    )(x, w)


_SCAT = 80
_POOL_CHUNK = 160
_ROWS_PER_TILE = S // NS
_POOL_NCHUNK = _ROWS_PER_TILE // _POOL_CHUNK
_NP = 10240
_SEG_PER_TILE = _NP // NS


def _pool_body(x_hbm, idx_hbm, zeros_hbm, pool0_hbm, pool1_hbm,
               table, rows0, rows1, ia0, ia1, ib0, ib1,
               rsem0, rsem1, iasem0, iasem1, ibsem0, ibsem1,
               scsem0, scsem1):
    c = lax.axis_index("c")
    s = lax.axis_index("s")
    pltpu.sync_copy(zeros_hbm, table.at[pl.ds(s * _SEG_PER_TILE, _SEG_PER_TILE)])
    plsc.subcore_barrier()

    rows = (rows0, rows1)
    ia = (ia0, ia1)
    ib = (ib0, ib1)
    rsems = (rsem0, rsem1)
    iasems = (iasem0, iasem1)
    ibsems = (ibsem0, ibsem1)
    scsems = (scsem0, scsem1)

    def issue(g, slot):
        base = s * _ROWS_PER_TILE + g * _POOL_CHUNK
        pltpu.async_copy(x_hbm.at[pl.ds(base, _POOL_CHUNK)], rows[slot],
                         rsems[slot])
        pltpu.async_copy(idx_hbm.at[pl.ds(c * S + base, _SCAT)],
                         ia[slot], iasems[slot])
        pltpu.async_copy(idx_hbm.at[pl.ds(c * S + base + _SCAT, _SCAT)],
                         ib[slot], ibsems[slot])

    def process(g, slot):
        pltpu.make_async_copy(x_hbm.at[pl.ds(0, _POOL_CHUNK)],
                              rows[slot], rsems[slot]).wait()
        pltpu.make_async_copy(idx_hbm.at[pl.ds(0, _SCAT)],
                              ia[slot], iasems[slot]).wait()
        pltpu.make_async_copy(idx_hbm.at[pl.ds(0, _SCAT)],
                              ib[slot], ibsems[slot]).wait()
        cpa = pltpu.async_copy(rows[slot].at[pl.ds(0, _SCAT)],
                               table.at[ia[slot]], scsems[slot], add=True)
        pltpu.sync_copy(rows[slot].at[pl.ds(_SCAT, _SCAT)],
                        table.at[ib[slot]], add=True)
        cpa.wait()

    issue(0, 0)
    issue(1, 1)

    def pair(g2, carry):
        for slot in range(2):
            g = g2 * 2 + slot
            process(g, slot)

            @pl.when(g + 2 < _POOL_NCHUNK)
            def _():
                issue(g + 2, slot)
        return carry

    lax.fori_loop(0, _POOL_NCHUNK // 2, pair, 0)
    process(_POOL_NCHUNK - 1, 0)
    plsc.subcore_barrier()
    stripe = pl.ds(s * _SEG_PER_TILE, _SEG_PER_TILE)

    @pl.when(c == 0)
    def _():
        pltpu.sync_copy(table.at[stripe], pool0_hbm.at[stripe])

    @pl.when(c == 1)
    def _():
        pltpu.sync_copy(table.at[stripe], pool1_hbm.at[stripe])


def _pools(x, idx2, zeros):
    mesh = plsc.VectorSubcoreMesh(core_axis_name="c", subcore_axis_name="s")
    return pl.kernel(
        _pool_body,
        out_type=(
            jax.ShapeDtypeStruct((_NP, D), jnp.float32),
            jax.ShapeDtypeStruct((_NP, D), jnp.float32),
        ),
        mesh=mesh,
        scratch_types=(
            [pltpu.VMEM_SHARED((_NP, D), jnp.float32)]
            + [pltpu.VMEM((_POOL_CHUNK, D), jnp.float32)] * 2
            + [pltpu.VMEM((_SCAT,), jnp.int32)] * 4
            + [pltpu.SemaphoreType.DMA] * 8
        ),
    )(x, idx2, zeros)


def _tables_body(rp_ref, cp_ref, ncol_ref, nrow_ref, nall_ref,
                 w1_ref, w2_ref, w3_ref, p1_ref, p2_ref):
    rp = rp_ref[...][:N]
    cp = cp_ref[...][:N]
    allp = jnp.sum(rp, axis=0, keepdims=True) / nall_ref[...]
    p3 = jnp.dot(allp, w3_ref[...], preferred_element_type=jnp.float32)
    p1_ref[...] = jnp.dot(rp / ncol_ref[...], w1_ref[...],
                          preferred_element_type=jnp.float32) + p3
    p2_ref[...] = jnp.dot(cp / nrow_ref[...], w2_ref[...],
                          preferred_element_type=jnp.float32)


def _tables(row_pool, col_pool, ncol, nrow, nall, w1, w2, w3):
    return pl.pallas_call(
        _tables_body,
        out_shape=(
            jax.ShapeDtypeStruct((N, D), jnp.float32),
            jax.ShapeDtypeStruct((N, D), jnp.float32),
        ),
    )(row_pool, col_pool, ncol, nrow, nall, w1, w2, w3)


_OUT_CHUNK = 80
_ROWS_PER_WORKER = S // (NC * NS)
_OUT_NCHUNK = _ROWS_PER_WORKER // _OUT_CHUNK


def _fuse_body(t_hbm, ic_hbm, ir_hbm, p1_hbm, p2_hbm, out_hbm,
               t0, t1, t2, g10, g11, g12, g20, g21, g22,
               ic0, ic1, ic2, ir0, ir1, ir2,
               tsem0, tsem1, tsem2, icsem0, icsem1, icsem2,
               irsem0, irsem1, irsem2, g1sem0, g1sem1, g1sem2,
               g2sem0, g2sem1, g2sem2):
    c = lax.axis_index("c")
    s = lax.axis_index("s")
    w = s * NC + c
    tb = (t0, t1, t2)
    g1b = (g10, g11, g12)
    g2b = (g20, g21, g22)
    icb = (ic0, ic1, ic2)
    irb = (ir0, ir1, ir2)
    tsems = (tsem0, tsem1, tsem2)
    icsems = (icsem0, icsem1, icsem2)
    irsems = (irsem0, irsem1, irsem2)
    g1sems = (g1sem0, g1sem1, g1sem2)
    g2sems = (g2sem0, g2sem1, g2sem2)

    def issue_loads(g, slot):
        base = w * _ROWS_PER_WORKER + g * _OUT_CHUNK
        pltpu.async_copy(ic_hbm.at[pl.ds(base, _OUT_CHUNK)], icb[slot],
                         icsems[slot])
        pltpu.async_copy(ir_hbm.at[pl.ds(base, _OUT_CHUNK)], irb[slot],
                         irsems[slot])
        pltpu.async_copy(t_hbm.at[pl.ds(base, _OUT_CHUNK)], tb[slot],
                         tsems[slot])

    def wait_idx(slot):
        pltpu.make_async_copy(ic_hbm.at[pl.ds(0, _OUT_CHUNK)], icb[slot],
                              icsems[slot]).wait()
        pltpu.make_async_copy(ir_hbm.at[pl.ds(0, _OUT_CHUNK)], irb[slot],
                              irsems[slot]).wait()

    def issue_gathers(slot):
        pltpu.async_copy(p1_hbm.at[icb[slot]], g1b[slot], g1sems[slot])
        pltpu.async_copy(p2_hbm.at[irb[slot]], g2b[slot], g2sems[slot])

    def wait_idx_issue_gathers(slot):
        wait_idx(slot)
        issue_gathers(slot)

    def wait_gathers_t(slot):
        pltpu.make_async_copy(p1_hbm.at[pl.ds(0, _OUT_CHUNK)], g1b[slot],
                              g1sems[slot]).wait()
        pltpu.make_async_copy(p2_hbm.at[pl.ds(0, _OUT_CHUNK)], g2b[slot],
                              g2sems[slot]).wait()
        pltpu.make_async_copy(t_hbm.at[pl.ds(0, _OUT_CHUNK)], tb[slot],
                              tsems[slot]).wait()

    def compute_store(g, slot):
        def row(r, cc):
            for k in range(D // LANES):
                sl = pl.ds(k * LANES, LANES)
                plsc.addupdate(tb[slot].at[r, sl],
                               g1b[slot][r, sl] + g2b[slot][r, sl])
            return cc

        lax.fori_loop(0, _OUT_CHUNK, row, 0)
        base = w * _ROWS_PER_WORKER + g * _OUT_CHUNK
        pltpu.sync_copy(tb[slot], out_hbm.at[pl.ds(base, _OUT_CHUNK)])

    issue_loads(0, 0)
    issue_loads(1, 1)
    issue_loads(2, 2)
    wait_idx_issue_gathers(0)
    wait_idx_issue_gathers(1)

    def triple(g3_, carry):
        for slot in range(3):
            g = g3_ * 3 + slot
            wait_gathers_t(slot)
            compute_store(g, slot)

            @pl.when(g + 3 < _OUT_NCHUNK)
            def _():
                issue_loads(g + 3, slot)

            @pl.when(g + 2 < _OUT_NCHUNK)
            def _():
                wait_idx_issue_gathers((slot + 2) % 3)
        return carry

    lax.fori_loop(0, _OUT_NCHUNK // 3, triple, 0)
    wait_gathers_t(0)
    compute_store(_OUT_NCHUNK - 2, 0)
    wait_gathers_t(1)
    compute_store(_OUT_NCHUNK - 1, 1)


def _fuse(t, idx_col, idx_row, p1, p2):
    mesh = plsc.VectorSubcoreMesh(core_axis_name="c", subcore_axis_name="s")
    return pl.kernel(
        _fuse_body,
        out_type=jax.ShapeDtypeStruct((S, D), jnp.float32),
        mesh=mesh,
        scratch_types=(
            [pltpu.VMEM((_OUT_CHUNK, D), jnp.float32)] * 9
            + [pltpu.VMEM((_OUT_CHUNK,), jnp.int32)] * 6
            + [pltpu.SemaphoreType.DMA] * 15
        ),
    )(t, idx_col, idx_row, p1, p2)


def kernel(input_layer, idx_col, idx_col_norm, idx_row, idx_row_norm,
           idx_all, idx_all_norm, W):
    idx2 = jnp.concatenate([idx_col, idx_row])
    zeros = jnp.zeros((_SEG_PER_TILE, D), jnp.float32)
    pool0, pool1 = _pools(input_layer, idx2, zeros)
    p1, p2 = _tables(pool0, pool1,
                     idx_col_norm.reshape(N, 1), idx_row_norm.reshape(N, 1),
                     idx_all_norm.reshape(1, 1), W[1], W[2], W[3])
    t = _matmul(input_layer, W[0])
    return _fuse(t, idx_col, idx_row, p1, p2)
